# baseline (speedup 1.0000x reference)
import functools

import jax
import jax.numpy as jnp
from jax import lax
from jax.experimental import pallas as pl
from jax.experimental.pallas import tpu as pltpu
from jax.experimental.pallas import tpu_sc as plsc

VOCAB = 100000
EMBED = 128
CONTEXT = 200
HIDDEN = 128

NW = 32
B_PAD = 256
B_PER_W = B_PAD // NW

K_BLK = 3200
V_BLK = 8192
NV = -(-VOCAB // V_BLK)


def _gather_sc(idx_pad, table):
    mesh = plsc.VectorSubcoreMesh(core_axis_name="c", subcore_axis_name="s")

    @functools.partial(
        pl.kernel,
        out_type=jax.ShapeDtypeStruct((B_PAD, EMBED), jnp.float32),
        mesh=mesh,
        scratch_types=[
            pltpu.VMEM((B_PER_W,), jnp.int32),
            pltpu.VMEM((B_PER_W, EMBED), jnp.float32),
            pltpu.SemaphoreType.DMA,
        ],
    )
    def gather(idx_hbm, table_hbm, out_hbm, idx_v, rows_v, sem):
        wid = lax.axis_index("s") * 2 + lax.axis_index("c")
        base = wid * B_PER_W
        pltpu.sync_copy(idx_hbm.at[pl.ds(base, B_PER_W)], idx_v)
        pltpu.async_copy(table_hbm.at[idx_v], rows_v, sem).wait()
        pltpu.sync_copy(rows_v, out_hbm.at[pl.ds(base, B_PER_W)])

    return gather(idx_pad, table)


def _layer1(e_flat, W1, b1_row):
    nk = (CONTEXT * EMBED) // K_BLK

    def body(e_ref, w_ref, b_ref, out_ref):
        i = pl.program_id(0)

        @pl.when(i == 0)
        def _():
            out_ref[...] = jnp.zeros_like(out_ref)

        out_ref[...] += jnp.dot(e_ref[...], w_ref[...],
                                preferred_element_type=jnp.float32)

        @pl.when(i == nk - 1)
        def _():
            out_ref[...] = jnp.maximum(out_ref[...] + b_ref[...], 0.0)

    return pl.pallas_call(
        body,
        grid=(nk,),
        in_specs=[
            pl.BlockSpec((1, K_BLK), lambda i: (0, i)),
            pl.BlockSpec((K_BLK, HIDDEN), lambda i: (i, 0)),
            pl.BlockSpec((1, HIDDEN), lambda i: (0, 0)),
        ],
        out_specs=pl.BlockSpec((1, HIDDEN), lambda i: (0, 0)),
        out_shape=jax.ShapeDtypeStruct((1, HIDDEN), jnp.float32),
    )(e_flat, W1, b1_row)


def _layer2_stats(h, W2, b2_row):
    def body(h_ref, w_ref, b_ref, logits_ref, norm_ref, m_ref, s_ref):
        i = pl.program_id(0)
        z = jnp.dot(h_ref[...], w_ref[...],
                    preferred_element_type=jnp.float32) + b_ref[...]
        logits_ref[...] = z
        col = i * V_BLK + lax.broadcasted_iota(jnp.int32, (1, V_BLK), 1)
        zm = jnp.where(col < VOCAB, z, -jnp.inf)
        bm = jnp.max(zm)

        @pl.when(i == 0)
        def _():
            m_ref[0] = bm
            s_ref[0] = jnp.sum(jnp.exp(zm - bm))

        @pl.when(i > 0)
        def _():
            m_old = m_ref[0]
            new_m = jnp.maximum(m_old, bm)
            s_ref[0] = s_ref[0] * jnp.exp(m_old - new_m) + \
                jnp.sum(jnp.exp(zm - new_m))
            m_ref[0] = new_m

        @pl.when(i == NV - 1)
        def _():
            norm_ref[...] = jnp.broadcast_to(
                m_ref[0] + jnp.log(s_ref[0]), (1, HIDDEN))

    return pl.pallas_call(
        body,
        grid=(NV,),
        in_specs=[
            pl.BlockSpec((1, HIDDEN), lambda i: (0, 0)),
            pl.BlockSpec((HIDDEN, V_BLK), lambda i: (0, i)),
            pl.BlockSpec((1, V_BLK), lambda i: (0, i)),
        ],
        out_specs=[
            pl.BlockSpec((1, V_BLK), lambda i: (0, i)),
            pl.BlockSpec((1, HIDDEN), lambda i: (0, 0)),
        ],
        out_shape=[
            jax.ShapeDtypeStruct((1, VOCAB), jnp.float32),
            jax.ShapeDtypeStruct((1, HIDDEN), jnp.float32),
        ],
        scratch_shapes=[pltpu.SMEM((1,), jnp.float32),
                        pltpu.SMEM((1,), jnp.float32)],
    )(h, W2, b2_row)


def _normalize(logits, norm):
    def body(l_ref, n_ref, o_ref):
        o_ref[...] = l_ref[...] - jnp.max(n_ref[...])

    return pl.pallas_call(
        body,
        out_shape=jax.ShapeDtypeStruct((1, VOCAB), jnp.float32),
    )(logits, norm)


def kernel(idx, table, W1, b1, W2, b2):
    idx_pad = jnp.zeros((B_PAD,), jnp.int32).at[:CONTEXT].set(
        idx.astype(jnp.int32))
    rows = _gather_sc(idx_pad, table)
    e_flat = rows[:CONTEXT].reshape(1, CONTEXT * EMBED)
    h = _layer1(e_flat, W1, b1.reshape(1, HIDDEN))
    logits, norm = _layer2_stats(h, W2, b2.reshape(1, VOCAB))
    return _normalize(logits, norm)


# speedup vs baseline: 1.0121x; 1.0121x over previous
import functools

import jax
import jax.numpy as jnp
from jax import lax
from jax.experimental import pallas as pl
from jax.experimental.pallas import tpu as pltpu
from jax.experimental.pallas import tpu_sc as plsc

VOCAB = 100000
EMBED = 128
CONTEXT = 200
HIDDEN = 128

B_PER_W = 8
NW_USED = CONTEXT // B_PER_W

K_BLK = 3200
NK = (CONTEXT * EMBED) // K_BLK
V_BLK = 8192
NV = -(-VOCAB // V_BLK)
P1 = NK
P2 = NK + NV
NSTEPS = NK + 2 * NV


def _gather_sc(idx, table):
    mesh = plsc.VectorSubcoreMesh(core_axis_name="c", subcore_axis_name="s")

    @functools.partial(
        pl.kernel,
        out_type=jax.ShapeDtypeStruct((CONTEXT, EMBED), jnp.float32),
        mesh=mesh,
        scratch_types=[
            pltpu.VMEM((B_PER_W,), jnp.int32),
            pltpu.VMEM((B_PER_W, EMBED), jnp.float32),
            pltpu.SemaphoreType.DMA,
        ],
    )
    def gather(idx_hbm, table_hbm, out_hbm, idx_v, rows_v, sem):
        wid = lax.axis_index("s") * 2 + lax.axis_index("c")

        @pl.when(wid < NW_USED)
        def _():
            base = wid * B_PER_W
            pltpu.sync_copy(idx_hbm.at[pl.ds(base, B_PER_W)], idx_v)
            pltpu.async_copy(table_hbm.at[idx_v], rows_v, sem).wait()
            pltpu.sync_copy(rows_v, out_hbm.at[pl.ds(base, B_PER_W)])

    return gather(idx, table)


def _fused_tc(e_flat, W1, b1_row, W2, b2_row):
    def body(e_ref, w1_ref, b1_ref, w2_ref, b2_ref, out_ref,
             acc_ref, logits_ref, m_ref, s_ref):
        i = pl.program_id(0)

        @pl.when(i < P1)
        def _():
            @pl.when(i == 0)
            def _():
                acc_ref[...] = jnp.zeros_like(acc_ref)

            acc_ref[...] += jnp.dot(e_ref[...], w1_ref[...],
                                    preferred_element_type=jnp.float32)

            @pl.when(i == P1 - 1)
            def _():
                acc_ref[...] = jnp.maximum(acc_ref[...] + b1_ref[...], 0.0)

        @pl.when((i >= P1) & (i < P2))
        def _():
            j = i - P1
            z = jnp.dot(acc_ref[...], w2_ref[...],
                        preferred_element_type=jnp.float32) + b2_ref[...]
            logits_ref[pl.ds(j, 1), :] = z
            col = j * V_BLK + lax.broadcasted_iota(jnp.int32, (1, V_BLK), 1)
            zm = jnp.where(col < VOCAB, z, -jnp.inf)
            bm = jnp.max(zm)

            @pl.when(j == 0)
            def _():
                m_ref[0] = bm
                s_ref[0] = jnp.sum(jnp.exp(zm - bm))

            @pl.when(j > 0)
            def _():
                m_old = m_ref[0]
                new_m = jnp.maximum(m_old, bm)
                s_ref[0] = s_ref[0] * jnp.exp(m_old - new_m) + \
                    jnp.sum(jnp.exp(zm - new_m))
                m_ref[0] = new_m

        @pl.when(i >= P2)
        def _():
            j = i - P2
            norm = m_ref[0] + jnp.log(s_ref[0])
            out_ref[...] = logits_ref[pl.ds(j, 1), :] - norm

    return pl.pallas_call(
        body,
        grid=(NSTEPS,),
        in_specs=[
            pl.BlockSpec((1, K_BLK),
                         lambda i: (0, jnp.minimum(i, P1 - 1))),
            pl.BlockSpec((K_BLK, HIDDEN),
                         lambda i: (jnp.minimum(i, P1 - 1), 0)),
            pl.BlockSpec((1, HIDDEN), lambda i: (0, 0)),
            pl.BlockSpec((HIDDEN, V_BLK),
                         lambda i: (0, jnp.clip(i - P1, 0, NV - 1))),
            pl.BlockSpec((1, V_BLK),
                         lambda i: (0, jnp.clip(i - P1, 0, NV - 1))),
        ],
        out_specs=pl.BlockSpec((1, V_BLK),
                               lambda i: (0, jnp.clip(i - P2, 0, NV - 1))),
        out_shape=jax.ShapeDtypeStruct((1, VOCAB), jnp.float32),
        scratch_shapes=[
            pltpu.VMEM((1, HIDDEN), jnp.float32),
            pltpu.VMEM((NV, V_BLK), jnp.float32),
            pltpu.SMEM((1,), jnp.float32),
            pltpu.SMEM((1,), jnp.float32),
        ],
    )(e_flat, W1, b1_row, W2, b2_row)


def kernel(idx, table, W1, b1, W2, b2):
    rows = _gather_sc(idx.astype(jnp.int32), table)
    e_flat = rows.reshape(1, CONTEXT * EMBED)
    return _fused_tc(e_flat, W1, b1.reshape(1, HIDDEN),
                     W2, b2.reshape(1, VOCAB))
